# Initial kernel scaffold; baseline (speedup 1.0000x reference)
import jax
import jax.numpy as jnp
from jax.experimental import pallas as pl


def kernel(image_x, all_edges, W_out, b_out):
    raise NotImplementedError("write your pallas kernel here")


# speedup vs baseline: 18.7571x; 18.7571x over previous
import jax
import jax.numpy as jnp
from jax.experimental import pallas as pl
from jax.experimental.pallas import tpu as pltpu

FS = 256
DT = 15.0
DT2 = DT * DT
INV_DT = 1.0 / DT
R = 16
NT = FS // R


def _body(ep_ref, img_ref, wimg_ref, wpn_ref, b_ref, out_ref, facc, pvec):
    i = pl.program_id(0)

    @pl.when(i == 0)
    def _():
        facc[...] = jnp.zeros((8, 128), jnp.float32)
        pvec[...] = jnp.zeros((16, 256), jnp.float32)

    y0 = (i * R).astype(jnp.float32)
    Vg = jax.lax.broadcasted_iota(jnp.int32, (R + 1, 256), 0).astype(jnp.float32) + y0
    Ug = jax.lax.broadcasted_iota(jnp.int32, (R + 1, 256), 1).astype(jnp.float32)
    valid = (Vg <= 255.0).astype(jnp.float32)

    w_rows = []
    for b in range(4):
        cnt = jnp.zeros((R + 1, 256), jnp.float32)
        spo = jnp.zeros((R + 1, 256), jnp.float32)
        snd = jnp.zeros((R + 1, 256), jnp.float32)
        for ei in range(32):
            p1v = ep_ref[b, ei, 0]
            p1u = ep_ref[b, ei, 1]
            p2v = ep_ref[b, ei, 2]
            p2u = ep_ref[b, ei, 3]
            n0 = ep_ref[b, ei, 4]
            n1 = ep_ref[b, ei, 5]
            c0 = ep_ref[b, ei, 6]
            c1 = ep_ref[b, ei, 7]
            r0 = Vg - p1v
            r1 = Ug - p1u
            nd = jnp.abs(r0 * n0 + r1 * n1)
            dd = r0 * c0 + r1 * c1
            q1 = r0 * r0 + r1 * r1
            s0 = Vg - p2v
            s1 = Ug - p2u
            q2 = s0 * s0 + s1 * s1
            m = ((nd <= DT) & (dd >= 0.0) & (dd <= 1.0)) \
                | (q1 <= DT2) | (q2 <= DT2)
            mf = m.astype(jnp.float32)
            cnt = cnt + mf
            spo = spo + mf * jnp.maximum(dd, 1.0 - dd)
            snd = snd + mf * (1.0 - nd * INV_DT)
        M = (cnt > 0.0).astype(jnp.float32) * valid
        den = jnp.maximum(cnt[:R], 1e-4)
        pvec[b : b + 1, :] += jnp.sum(M[:R], axis=0, keepdims=True)
        pvec[4 + b : 5 + b, :] += jnp.sum(spo[:R] / den, axis=0, keepdims=True)
        pvec[8 + b : 9 + b, :] += jnp.sum(snd[:R] / den, axis=0, keepdims=True)
        A = M[:R] + M[1 : R + 1]
        Ax = jnp.concatenate([A[:, 1:], jnp.zeros((R, 1), jnp.float32)], axis=1)
        w_rows.append(((A + Ax) * 0.25).reshape(1, R * 256))

    w4 = jnp.concatenate(w_rows, axis=0)
    w8 = jnp.concatenate([w4, jnp.zeros((4, R * 256), jnp.float32)], axis=0)
    facc[...] += jax.lax.dot_general(
        w8, img_ref[...], (((1,), (1,)), ((), ())),
        preferred_element_type=jnp.float32)

    @pl.when(i == NT - 1)
    def _():
        cntb = jnp.sum(pvec[0:4, :], axis=1, keepdims=True)
        spob = jnp.sum(pvec[4:8, :], axis=1, keepdims=True)
        sndb = jnp.sum(pvec[8:12, :], axis=1, keepdims=True)
        z = jax.lax.dot_general(
            facc[...], wimg_ref[...], (((1,), (0,)), ((), ())),
            preferred_element_type=jnp.float32)[0:4]
        z = z + spob * wpn_ref[0:1, :] + sndb * wpn_ref[1:2, :]
        den = jnp.maximum(cntb, 1.0)
        out_ref[...] = jnp.maximum(z / den + b_ref[...], 0.0)


def kernel(image_x, all_edges, W_out, b_out):
    img2d = jnp.reshape(image_x, (128, FS * FS))
    e = all_edges * float(FS)
    p1v, p1u, p2v, p2u = e[..., 0], e[..., 1], e[..., 2], e[..., 3]
    dv = p2v - p1v
    du = p2u - p1u
    L = jnp.maximum(jnp.sqrt(dv * dv + du * du), 1e-4)
    dir0 = dv / L
    dir1 = du / L
    invL = 1.0 / L
    ep = jnp.stack(
        [p1v, p1u, p2v, p2u, dir1, -dir0, dir0 * invL, dir1 * invL], axis=-1)

    wimg = W_out[:128]
    wpn = W_out[128:130]
    brow = jnp.reshape(b_out, (1, 256))

    return pl.pallas_call(
        _body,
        grid=(NT,),
        in_specs=[
            pl.BlockSpec(memory_space=pltpu.SMEM),
            pl.BlockSpec((128, R * 256), lambda i: (0, i)),
            pl.BlockSpec((128, 256), lambda i: (0, 0)),
            pl.BlockSpec((2, 256), lambda i: (0, 0)),
            pl.BlockSpec((1, 256), lambda i: (0, 0)),
        ],
        out_specs=pl.BlockSpec((4, 256), lambda i: (0, 0)),
        out_shape=jax.ShapeDtypeStruct((4, 256), jnp.float32),
        scratch_shapes=[
            pltpu.VMEM((8, 128), jnp.float32),
            pltpu.VMEM((16, 256), jnp.float32),
        ],
    )(ep, img2d, wimg, wpn, brow)
